# Initial kernel scaffold; baseline (speedup 1.0000x reference)
import jax
import jax.numpy as jnp
from jax.experimental import pallas as pl


def kernel(level, tag, level_table, tag_table, W1, b1, W2, b2):
    raise NotImplementedError("write your pallas kernel here")


# speedup vs baseline: 3.7224x; 3.7224x over previous
import functools

import jax
import jax.numpy as jnp
from jax import lax
from jax.experimental import pallas as pl
from jax.experimental.pallas import tpu as pltpu
from jax.experimental.pallas import tpu_sc as plsc

_NW = 32
_K = 128


def _mm_bias_kernel(x_ref, w_ref, b_ref, o_ref):
    o_ref[...] = (
        jnp.dot(x_ref[...], w_ref[...], preferred_element_type=jnp.float32)
        + b_ref[...]
    )


def _table_matmul(table, w, bias):
    V, D = table.shape
    Do = w.shape[1]
    BT = 2000 if V % 2000 == 0 else V
    return pl.pallas_call(
        _mm_bias_kernel,
        grid=(V // BT,),
        in_specs=[
            pl.BlockSpec((BT, D), lambda i: (i, 0)),
            pl.BlockSpec((D, Do), lambda i: (0, 0)),
            pl.BlockSpec((1, Do), lambda i: (0, 0)),
        ],
        out_specs=pl.BlockSpec((BT, Do), lambda i: (i, 0)),
        out_shape=jax.ShapeDtypeStruct((V, Do), jnp.float32),
    )(table, w, bias.reshape(1, Do))


def _relu_mm_kernel(h_ref, w_ref, b_ref, o_ref):
    h = jnp.maximum(h_ref[...], 0.0)
    o_ref[...] = (
        jnp.dot(h, w_ref[...], preferred_element_type=jnp.float32) + b_ref[...]
    )


def _relu_matmul(h_pre, w, bias):
    N, D = h_pre.shape
    Do = w.shape[1]
    BR = 2048
    return pl.pallas_call(
        _relu_mm_kernel,
        grid=(N // BR,),
        in_specs=[
            pl.BlockSpec((BR, D), lambda i: (i, 0)),
            pl.BlockSpec((D, Do), lambda i: (0, 0)),
            pl.BlockSpec((1, Do), lambda i: (0, 0)),
        ],
        out_specs=pl.BlockSpec((BR, Do), lambda i: (i, 0)),
        out_shape=jax.ShapeDtypeStruct((N, Do), jnp.float32),
    )(h_pre, w, bias.reshape(1, Do))


def _gather_add(lvl_idx, tag_idx, g_lvl, g_tag):
    N = lvl_idx.shape[0]
    D = g_lvl.shape[1]
    assert N % (_NW * _K) == 0
    per_w = N // _NW
    nchunk = per_w // _K
    nv = D // 16

    mesh = plsc.VectorSubcoreMesh(core_axis_name="c", subcore_axis_name="s")

    def body(lvl_hbm, tag_hbm, gl_hbm, gt_hbm, out_hbm,
             il0, il1, it0, it1, rl0, rl1, rt0, rt1,
             sl0, sl1, st0, st1):
        ils = (il0, il1)
        its = (it0, it1)
        rls = (rl0, rl1)
        rts = (rt0, rt1)
        sls = (sl0, sl1)
        sts = (st0, st1)
        w = lax.axis_index("s") * 2 + lax.axis_index("c")
        wbase = w * per_w

        def issue(c, b):
            base = wbase + c * _K
            pltpu.sync_copy(lvl_hbm.at[pl.ds(base, _K)], ils[b])
            pltpu.sync_copy(tag_hbm.at[pl.ds(base, _K)], its[b])
            pltpu.async_copy(gl_hbm.at[ils[b]], rls[b], sls[b])
            pltpu.async_copy(gt_hbm.at[its[b]], rts[b], sts[b])

        issue(0, 0)
        issue(1, 1)

        @pl.loop(0, nchunk, step=2)
        def _chunks(g):
            for b in range(2):
                c = g + b
                pltpu.make_async_copy(gl_hbm.at[ils[b]], rls[b], sls[b]).wait()
                pltpu.make_async_copy(gt_hbm.at[its[b]], rts[b], sts[b]).wait()

                @pl.loop(0, _K)
                def _rows(r):
                    for v in range(nv):
                        sl = pl.ds(v * 16, 16)
                        rts[b][r, sl] = rls[b][r, sl] + rts[b][r, sl]

                pltpu.sync_copy(rts[b], out_hbm.at[pl.ds(wbase + c * _K, _K)])

                @pl.when(c + 2 < nchunk)
                def _():
                    issue(c + 2, b)

    fn = pl.kernel(
        body,
        out_type=jax.ShapeDtypeStruct((N, D), jnp.float32),
        mesh=mesh,
        scratch_types=[
            pltpu.VMEM((_K,), jnp.int32),
            pltpu.VMEM((_K,), jnp.int32),
            pltpu.VMEM((_K,), jnp.int32),
            pltpu.VMEM((_K,), jnp.int32),
            pltpu.VMEM((_K, D), jnp.float32),
            pltpu.VMEM((_K, D), jnp.float32),
            pltpu.VMEM((_K, D), jnp.float32),
            pltpu.VMEM((_K, D), jnp.float32),
            pltpu.SemaphoreType.DMA,
            pltpu.SemaphoreType.DMA,
            pltpu.SemaphoreType.DMA,
            pltpu.SemaphoreType.DMA,
        ],
    )
    return fn(lvl_idx, tag_idx, g_lvl, g_tag)


def kernel(level, tag, level_table, tag_table, W1, b1, W2, b2):
    B, L = level.shape
    D = level_table.shape[1]
    N = B * L
    g_lvl = _table_matmul(level_table, W1[:D], b1)
    g_tag = _table_matmul(tag_table, W1[D:], jnp.zeros((D,), jnp.float32))
    h_pre = _gather_add(level.reshape(N), tag.reshape(N), g_lvl, g_tag)
    out = _relu_matmul(h_pre, W2, b2)
    return out.reshape(B, L, D)


# speedup vs baseline: 4.0943x; 1.0999x over previous
import functools

import jax
import jax.numpy as jnp
from jax import lax
from jax.experimental import pallas as pl
from jax.experimental.pallas import tpu as pltpu
from jax.experimental.pallas import tpu_sc as plsc

_NW = 32
_K = 128


def _mm_bias_kernel(x_ref, w_ref, b_ref, o_ref):
    o_ref[...] = (
        jnp.dot(x_ref[...], w_ref[...], preferred_element_type=jnp.float32)
        + b_ref[...]
    )


def _table_matmul(table, w, bias):
    V, D = table.shape
    Do = w.shape[1]
    BT = 2000 if V % 2000 == 0 else V
    return pl.pallas_call(
        _mm_bias_kernel,
        grid=(V // BT,),
        in_specs=[
            pl.BlockSpec((BT, D), lambda i: (i, 0)),
            pl.BlockSpec((D, Do), lambda i: (0, 0)),
            pl.BlockSpec((1, Do), lambda i: (0, 0)),
        ],
        out_specs=pl.BlockSpec((BT, Do), lambda i: (i, 0)),
        out_shape=jax.ShapeDtypeStruct((V, Do), jnp.float32),
    )(table, w, bias.reshape(1, Do))


def _relu_mm3d_kernel(L, BB, h_ref, w_ref, b_ref, o_ref):
    h = jnp.maximum(h_ref[...], 0.0)
    y = jnp.dot(h, w_ref[...], preferred_element_type=jnp.float32) + b_ref[...]
    for j in range(BB):
        o_ref[j] = y[j * L:(j + 1) * L, :]


def _relu_matmul(h_pre, w, bias, B, L):
    N, D = h_pre.shape
    Do = w.shape[1]
    BB = 16
    return pl.pallas_call(
        functools.partial(_relu_mm3d_kernel, L, BB),
        grid=(B // BB,),
        in_specs=[
            pl.BlockSpec((BB * L, D), lambda i: (i, 0)),
            pl.BlockSpec((D, Do), lambda i: (0, 0)),
            pl.BlockSpec((1, Do), lambda i: (0, 0)),
        ],
        out_specs=pl.BlockSpec((BB, L, Do), lambda i: (i, 0, 0)),
        out_shape=jax.ShapeDtypeStruct((B, L, Do), jnp.float32),
    )(h_pre, w, bias.reshape(1, Do))


def _gather_add(lvl_idx, tag_idx, g_lvl, g_tag):
    N = lvl_idx.shape[0]
    D = g_lvl.shape[1]
    assert N % (_NW * _K) == 0
    per_w = N // _NW
    nchunk = per_w // _K
    nv = D // 16

    mesh = plsc.VectorSubcoreMesh(core_axis_name="c", subcore_axis_name="s")

    def body(lvl_hbm, tag_hbm, gl_hbm, gt_hbm, out_hbm,
             il0, il1, it0, it1, rl0, rl1, rt0, rt1,
             sl0, sl1, st0, st1):
        ils = (il0, il1)
        its = (it0, it1)
        rls = (rl0, rl1)
        rts = (rt0, rt1)
        sls = (sl0, sl1)
        sts = (st0, st1)
        w = lax.axis_index("s") * 2 + lax.axis_index("c")
        wbase = w * per_w

        def issue(c, b):
            base = wbase + c * _K
            pltpu.sync_copy(lvl_hbm.at[pl.ds(base, _K)], ils[b])
            pltpu.sync_copy(tag_hbm.at[pl.ds(base, _K)], its[b])
            pltpu.async_copy(gl_hbm.at[ils[b]], rls[b], sls[b])
            pltpu.async_copy(gt_hbm.at[its[b]], rts[b], sts[b])

        issue(0, 0)
        issue(1, 1)

        @pl.loop(0, nchunk, step=2)
        def _chunks(g):
            for b in range(2):
                c = g + b
                pltpu.make_async_copy(gl_hbm.at[ils[b]], rls[b], sls[b]).wait()
                pltpu.make_async_copy(gt_hbm.at[its[b]], rts[b], sts[b]).wait()

                @pl.loop(0, _K)
                def _rows(r):
                    for v in range(nv):
                        sl = pl.ds(v * 16, 16)
                        rts[b][r, sl] = rls[b][r, sl] + rts[b][r, sl]

                pltpu.sync_copy(rts[b], out_hbm.at[pl.ds(wbase + c * _K, _K)])

                @pl.when(c + 2 < nchunk)
                def _():
                    issue(c + 2, b)

    fn = pl.kernel(
        body,
        out_type=jax.ShapeDtypeStruct((N, D), jnp.float32),
        mesh=mesh,
        scratch_types=[
            pltpu.VMEM((_K,), jnp.int32),
            pltpu.VMEM((_K,), jnp.int32),
            pltpu.VMEM((_K,), jnp.int32),
            pltpu.VMEM((_K,), jnp.int32),
            pltpu.VMEM((_K, D), jnp.float32),
            pltpu.VMEM((_K, D), jnp.float32),
            pltpu.VMEM((_K, D), jnp.float32),
            pltpu.VMEM((_K, D), jnp.float32),
            pltpu.SemaphoreType.DMA,
            pltpu.SemaphoreType.DMA,
            pltpu.SemaphoreType.DMA,
            pltpu.SemaphoreType.DMA,
        ],
    )
    return fn(lvl_idx, tag_idx, g_lvl, g_tag)


def kernel(level, tag, level_table, tag_table, W1, b1, W2, b2):
    B, L = level.shape
    D = level_table.shape[1]
    N = B * L
    g_lvl = _table_matmul(level_table, W1[:D], b1)
    g_tag = _table_matmul(tag_table, W1[D:], jnp.zeros((D,), jnp.float32))
    h_pre = _gather_add(level.reshape(N), tag.reshape(N), g_lvl, g_tag)
    return _relu_matmul(h_pre, W2, b2, B, L)


# speedup vs baseline: 6.3201x; 1.5436x over previous
import functools

import jax
import jax.numpy as jnp
from jax import lax
from jax.experimental import pallas as pl
from jax.experimental.pallas import tpu as pltpu
from jax.experimental.pallas import tpu_sc as plsc

_NW = 32
_K = 128


def _mm_bias_kernel(x_ref, w_ref, b_ref, o_ref):
    o_ref[...] = (
        jnp.dot(x_ref[...], w_ref[...], preferred_element_type=jnp.float32)
        + b_ref[...]
    )


def _table_matmul(table, w, bias):
    V, D = table.shape
    Do = w.shape[1]
    BT = 2000 if V % 2000 == 0 else V
    return pl.pallas_call(
        _mm_bias_kernel,
        grid=(V // BT,),
        in_specs=[
            pl.BlockSpec((BT, D), lambda i: (i, 0)),
            pl.BlockSpec((D, Do), lambda i: (0, 0)),
            pl.BlockSpec((1, Do), lambda i: (0, 0)),
        ],
        out_specs=pl.BlockSpec((BT, Do), lambda i: (i, 0)),
        out_shape=jax.ShapeDtypeStruct((V, Do), jnp.float32),
    )(table, w, bias.reshape(1, Do))


def _relu_mm3d_kernel(h_ref, w_ref, b_ref, o_ref):
    h = jnp.maximum(h_ref[...], 0.0)
    o_ref[0] = (
        jnp.dot(h, w_ref[...], preferred_element_type=jnp.float32) + b_ref[...]
    )


def _relu_matmul(h_lm, w, bias, B, L):
    N, D = h_lm.shape
    Do = w.shape[1]
    BR = 2048
    nb = B // BR
    return pl.pallas_call(
        _relu_mm3d_kernel,
        grid=(L, nb),
        in_specs=[
            pl.BlockSpec((BR, D), lambda l, i: (l * nb + i, 0)),
            pl.BlockSpec((D, Do), lambda l, i: (0, 0)),
            pl.BlockSpec((1, Do), lambda l, i: (0, 0)),
        ],
        out_specs=pl.BlockSpec((1, BR, Do), lambda l, i: (l, i, 0)),
        out_shape=jax.ShapeDtypeStruct((L, B, Do), jnp.float32),
    )(h_lm, w, bias.reshape(1, Do))


def _gather_add(lvl_idx, tag_idx, g_lvl, g_tag):
    N = lvl_idx.shape[0]
    D = g_lvl.shape[1]
    assert N % (_NW * _K) == 0
    per_w = N // _NW
    nchunk = per_w // _K
    nv = D // 16

    mesh = plsc.VectorSubcoreMesh(core_axis_name="c", subcore_axis_name="s")

    def body(lvl_hbm, tag_hbm, gl_hbm, gt_hbm, out_hbm,
             il0, il1, it0, it1, rl0, rl1, rt0, rt1,
             sl0, sl1, st0, st1):
        ils = (il0, il1)
        its = (it0, it1)
        rls = (rl0, rl1)
        rts = (rt0, rt1)
        sls = (sl0, sl1)
        sts = (st0, st1)
        w = lax.axis_index("s") * 2 + lax.axis_index("c")
        wbase = w * per_w

        def issue(c, b):
            base = wbase + c * _K
            pltpu.sync_copy(lvl_hbm.at[pl.ds(base, _K)], ils[b])
            pltpu.sync_copy(tag_hbm.at[pl.ds(base, _K)], its[b])
            pltpu.async_copy(gl_hbm.at[ils[b]], rls[b], sls[b])
            pltpu.async_copy(gt_hbm.at[its[b]], rts[b], sts[b])

        issue(0, 0)
        issue(1, 1)

        @pl.loop(0, nchunk, step=2)
        def _chunks(g):
            for b in range(2):
                c = g + b
                pltpu.make_async_copy(gl_hbm.at[ils[b]], rls[b], sls[b]).wait()
                pltpu.make_async_copy(gt_hbm.at[its[b]], rts[b], sts[b]).wait()

                @pl.loop(0, _K)
                def _rows(r):
                    for v in range(nv):
                        sl = pl.ds(v * 16, 16)
                        rts[b][r, sl] = rls[b][r, sl] + rts[b][r, sl]

                pltpu.sync_copy(rts[b], out_hbm.at[pl.ds(wbase + c * _K, _K)])

                @pl.when(c + 2 < nchunk)
                def _():
                    issue(c + 2, b)

    fn = pl.kernel(
        body,
        out_type=jax.ShapeDtypeStruct((N, D), jnp.float32),
        mesh=mesh,
        scratch_types=[
            pltpu.VMEM((_K,), jnp.int32),
            pltpu.VMEM((_K,), jnp.int32),
            pltpu.VMEM((_K,), jnp.int32),
            pltpu.VMEM((_K,), jnp.int32),
            pltpu.VMEM((_K, D), jnp.float32),
            pltpu.VMEM((_K, D), jnp.float32),
            pltpu.VMEM((_K, D), jnp.float32),
            pltpu.VMEM((_K, D), jnp.float32),
            pltpu.SemaphoreType.DMA,
            pltpu.SemaphoreType.DMA,
            pltpu.SemaphoreType.DMA,
            pltpu.SemaphoreType.DMA,
        ],
    )
    return fn(lvl_idx, tag_idx, g_lvl, g_tag)


def kernel(level, tag, level_table, tag_table, W1, b1, W2, b2):
    B, L = level.shape
    D = level_table.shape[1]
    N = B * L
    g_lvl = _table_matmul(level_table, W1[:D], b1)
    g_tag = _table_matmul(tag_table, W1[D:], jnp.zeros((D,), jnp.float32))
    lvl_lm = jnp.transpose(level).reshape(N)
    tag_lm = jnp.transpose(tag).reshape(N)
    h_lm = _gather_add(lvl_lm, tag_lm, g_lvl, g_tag)
    out_lm = _relu_matmul(h_lm, W2, b2, B, L)
    return jnp.transpose(out_lm, (1, 0, 2))
